# baseline (speedup 1.0000x reference)
import functools

import jax
import jax.numpy as jnp
from jax import lax
from jax.experimental import pallas as pl
from jax.experimental.pallas import tpu as pltpu
from jax.experimental.pallas import tpu_sc as plsc

B = 16384
D = 64
NC = 2
NS = 16
NW = NC * NS
BPW = B // NW
NCHUNK = 4
CHUNK = BPW // NCHUNK
L = 16
NGRP = BPW // L

_mesh = plsc.VectorSubcoreMesh(core_axis_name="c", subcore_axis_name="s")


def _gmf_body(users_ref, items_ref, ut_ref, it_ref, wb_ref, out_ref,
              idx_u, idx_i, rows_u, rows_i, out_v, wb_v, sem_u, sem_i):
    c = lax.axis_index("c")
    s = lax.axis_index("s")
    wid = s * NC + c
    base = wid * BPW

    pltpu.sync_copy(users_ref.at[wid], idx_u)
    pltpu.sync_copy(items_ref.at[wid], idx_i)
    pltpu.sync_copy(wb_ref, wb_v)

    copies = []
    for j in range(NCHUNK):
        copies.append(pltpu.async_copy(
            ut_ref.at[idx_u.at[j]], rows_u.at[pl.ds(j * CHUNK, CHUNK)], sem_u))
        copies.append(pltpu.async_copy(
            it_ref.at[idx_i.at[j]], rows_i.at[pl.ds(j * CHUNK, CHUNK)], sem_i))
    for cp in copies:
        cp.wait()

    wvecs = [wb_v[pl.ds(k * L, L)] for k in range(D // L)]
    wscal = [wvecs[d // L][d % L] for d in range(D)]
    bias_vec = wb_v[pl.ds(D, L)]
    iota = lax.iota(jnp.int32, L)
    cols = [jnp.full((L,), d, jnp.int32) for d in range(D)]

    def group_body(g, carry):
        row_idx = g * L + iota
        acc = jnp.zeros((L,), jnp.float32)
        for d in range(D):
            u = plsc.load_gather(rows_u, [row_idx, cols[d]])
            v = plsc.load_gather(rows_i, [row_idx, cols[d]])
            acc = acc + (u * v) * wscal[d]
        x = acc + bias_vec
        out_v[pl.ds(g * L, L)] = 1.0 / (1.0 + jnp.exp(-x))
        return carry

    lax.fori_loop(0, NGRP, group_body, 0)

    pltpu.sync_copy(out_v, out_ref.at[pl.ds(base, BPW)])


_gmf = functools.partial(
    pl.kernel,
    out_type=jax.ShapeDtypeStruct((B,), jnp.float32),
    mesh=_mesh,
    compiler_params=pltpu.CompilerParams(
        needs_layout_passes=False, use_tc_tiling_on_sc=False),
    scratch_types=[
        pltpu.VMEM((NCHUNK, CHUNK), jnp.int32),
        pltpu.VMEM((NCHUNK, CHUNK), jnp.int32),
        pltpu.VMEM((BPW, D), jnp.float32),
        pltpu.VMEM((BPW, D), jnp.float32),
        pltpu.VMEM((BPW,), jnp.float32),
        pltpu.VMEM((D + L,), jnp.float32),
        pltpu.SemaphoreType.DMA,
        pltpu.SemaphoreType.DMA,
    ],
)(_gmf_body)


def kernel(users, items, user_table, item_table, predict_w, predict_b):
    users2 = users.astype(jnp.int32).reshape(NW, NCHUNK, CHUNK)
    items2 = items.astype(jnp.int32).reshape(NW, NCHUNK, CHUNK)
    wb = jnp.concatenate(
        [predict_w.reshape(-1), jnp.full((L,), predict_b[0], jnp.float32)])
    return _gmf(users2, items2, user_table, item_table, wb)
